# Initial kernel scaffold; baseline (speedup 1.0000x reference)
import jax
import jax.numpy as jnp
from jax.experimental import pallas as pl


def kernel(example, label, mus_label, mus_class, noise_even, noise_odd):
    raise NotImplementedError("write your pallas kernel here")


# speedup vs baseline: 14.4215x; 14.4215x over previous
import jax
import jax.numpy as jnp
import numpy as np
from jax import lax
from jax.experimental import pallas as pl
from jax.experimental.pallas import tpu as pltpu
from jax.experimental.pallas import tpu_sc as plsc

S = 1024
NMAX = 64
D = 128
N_PAIRS = 63
N_EX = 2 * N_PAIRS
N_LAB = N_PAIRS - 1
SEQ_LEN = 188
FEAT = 2 * NMAX + D
EPS = 0.1
A = float(1.0 / np.sqrt(1.0 + EPS * EPS))
B = float(A * EPS / np.sqrt(D))

NC, NS = 2, 16
NW = NC * NS
ITEMS = S // NW
LANES = 16
NVH = D // LANES


def _sc_body(example_h, label_h, mus_label_h, mus_class_h, ne_h, no_h,
             out_h, eidx, lidx, crows, lrows, ne, no, buf,
             sem0, sem1, sem2, sem3):
    wid = lax.axis_index("s") * NC + lax.axis_index("c")
    base = wid * ITEMS

    def zrow(r, _):
        for j in range(NVH):
            buf[r, pl.ds(j * LANES, LANES)] = jnp.zeros((LANES,), jnp.float32)
        return 0
    lax.fori_loop(0, SEQ_LEN, zrow, 0)

    pltpu.sync_copy(example_h.at[pl.ds(base, ITEMS)], eidx)
    pltpu.sync_copy(label_h.at[pl.ds(base, ITEMS)], lidx)

    def item(i, _):
        s = base + i
        cg = pltpu.async_copy(mus_class_h.at[eidx.at[i]], crows, sem0)
        lg = pltpu.async_copy(mus_label_h.at[lidx.at[i]], lrows, sem1)
        nc_ = pltpu.async_copy(ne_h.at[s], ne, sem2)
        no_ = pltpu.async_copy(no_h.at[s], no, sem3)
        cg.wait()
        lg.wait()
        nc_.wait()
        no_.wait()

        def pair(p, _):
            for j in range(NVH):
                src = pl.ds(j * LANES, LANES)
                dst = pl.ds(D + j * LANES, LANES)
                buf[3 * p, dst] = A * crows[2 * p, src] + B * ne[p, src]
                buf[3 * p + 1, dst] = A * crows[2 * p + 1, src] + B * no[p, src]
            return 0
        lax.fori_loop(0, N_PAIRS, pair, 0)

        def labrow(p, _):
            for j in range(NVH):
                buf[3 * p + 2, pl.ds(D + j * LANES, LANES)] = \
                    lrows[p, pl.ds(j * LANES, LANES)]
            return 0
        lax.fori_loop(0, N_LAB, labrow, 0)

        pltpu.sync_copy(buf, out_h.at[s])
        return 0
    lax.fori_loop(0, ITEMS, item, 0)


def kernel(example, label, mus_label, mus_class, noise_even, noise_odd):
    mesh = plsc.VectorSubcoreMesh(core_axis_name="c", subcore_axis_name="s",
                                  num_cores=NC, num_subcores=NS)
    call = pl.kernel(
        _sc_body, mesh=mesh,
        out_type=jax.ShapeDtypeStruct((S, SEQ_LEN, FEAT), jnp.float32),
        scratch_types=[
            pltpu.VMEM((ITEMS, N_EX), jnp.int32),
            pltpu.VMEM((ITEMS, N_PAIRS), jnp.int32),
            pltpu.VMEM((N_EX, D), jnp.float32),
            pltpu.VMEM((N_PAIRS, D), jnp.float32),
            pltpu.VMEM((N_PAIRS, D), jnp.float32),
            pltpu.VMEM((N_PAIRS, D), jnp.float32),
            pltpu.VMEM((SEQ_LEN, FEAT), jnp.float32),
            pltpu.SemaphoreType.DMA,
            pltpu.SemaphoreType.DMA,
            pltpu.SemaphoreType.DMA,
            pltpu.SemaphoreType.DMA,
        ],
    )
    return call(example.astype(jnp.int32), label.astype(jnp.int32),
                mus_label, mus_class, noise_even, noise_odd)


# speedup vs baseline: 17.0928x; 1.1852x over previous
import jax
import jax.numpy as jnp
import numpy as np
from jax import lax
from jax.experimental import pallas as pl
from jax.experimental.pallas import tpu as pltpu
from jax.experimental.pallas import tpu_sc as plsc

S = 1024
NMAX = 64
D = 128
N_PAIRS = 63
N_EX = 2 * N_PAIRS
SEQ_LEN = 188
FEAT = 2 * NMAX + D
EPS = 0.1
A = float(1.0 / np.sqrt(1.0 + EPS * EPS))
B = float(A * EPS / np.sqrt(D))

NC, NS = 2, 16
NW = NC * NS
ITEMS = S // NW
LANES = 16
NVH = D // LANES

NP = (32, 31)
NL = (32, 30)
RH = (96, 92)


def _sc_body(example_h, label_h, mus_label_h, mus_class_h, ne_h, no_h,
             out_h, estage, lstage, eidx3, lidx3,
             crows0, lrows0, ne0, no0, crows1, lrows1, ne1, no1,
             buf0, buf1, zbuf, isem0, isem1, osem0, osem1):
    wid = lax.axis_index("s") * NC + lax.axis_index("c")
    base = wid * ITEMS

    def zrow(r, _):
        for j in range(NVH):
            zbuf[r, pl.ds(LANES * j, LANES)] = jnp.zeros((LANES,), jnp.float32)
        return 0
    lax.fori_loop(0, 96, zrow, 0)

    pltpu.sync_copy(example_h.at[pl.ds(base, ITEMS)], estage)
    pltpu.sync_copy(label_h.at[pl.ds(base, ITEMS)], lstage)
    zi = jnp.zeros((LANES,), jnp.int32)

    def rp(i, _):
        for j in range(4):
            eidx3[i, 0, pl.ds(16 * j, 16)] = estage[i, pl.ds(16 * j, 16)]
        eidx3[i, 1, pl.ds(48, 16)] = zi
        for src, dst in ((64, 0), (80, 16), (96, 32), (110, 46)):
            eidx3[i, 1, pl.ds(dst, 16)] = estage[i, pl.ds(src, 16)]
        for j in range(2):
            lidx3[i, 0, pl.ds(16 * j, 16)] = lstage[i, pl.ds(16 * j, 16)]
        lidx3[i, 1, pl.ds(16, 16)] = zi
        lidx3[i, 1, pl.ds(0, 16)] = lstage[i, pl.ds(32, 16)]
        lidx3[i, 1, pl.ds(15, 16)] = lstage[i, pl.ds(47, 16)]
        return 0
    lax.fori_loop(0, ITEMS, rp, 0)

    def in_copies(i, h, crows, lrows, ne, no, sem):
        s = base + i
        nph = NP[h]
        return (
            pltpu.make_async_copy(mus_class_h.at[eidx3.at[i, h]], crows, sem),
            pltpu.make_async_copy(mus_label_h.at[lidx3.at[i, h]], lrows, sem),
            pltpu.make_async_copy(ne_h.at[s, pl.ds(32 * h, nph)],
                                  ne.at[pl.ds(0, nph)], sem),
            pltpu.make_async_copy(no_h.at[s, pl.ds(32 * h, nph)],
                                  no.at[pl.ds(0, nph)], sem),
        )

    def out_copies(i, h, buf, sem):
        s = base + i
        rh = RH[h]
        rows = pl.ds(96 * h, rh)
        return (
            pltpu.make_async_copy(zbuf.at[pl.ds(0, rh)],
                                  out_h.at[s, rows, pl.ds(0, D)], sem),
            pltpu.make_async_copy(buf.at[pl.ds(0, rh)],
                                  out_h.at[s, rows, pl.ds(D, D)], sem),
        )

    def issue(copies):
        for c in copies:
            c.start()

    def drain(copies):
        for c in copies:
            c.wait()

    def compute(buf, crows, lrows, ne, no, h):
        def pair(q, _):
            for j in range(NVH):
                sj = pl.ds(LANES * j, LANES)
                buf[3 * q, sj] = A * crows[2 * q, sj] + B * ne[q, sj]
                buf[3 * q + 1, sj] = A * crows[2 * q + 1, sj] + B * no[q, sj]
            return 0
        lax.fori_loop(0, NP[h], pair, 0)

        def lab(l, _):
            for j in range(NVH):
                sj = pl.ds(LANES * j, LANES)
                buf[3 * l + 2, sj] = lrows[l, sj]
            return 0
        lax.fori_loop(0, NL[h], lab, 0)

    in0 = lambda i: in_copies(i, 0, crows0, lrows0, ne0, no0, isem0)
    in1 = lambda i: in_copies(i, 1, crows1, lrows1, ne1, no1, isem1)
    out0 = lambda i: out_copies(i, 0, buf0, osem0)
    out1 = lambda i: out_copies(i, 1, buf1, osem1)

    issue(in0(0))
    issue(in1(0))

    def step(t, _):
        @pl.when(t > 0)
        def _():
            drain(out0(t - 1))
        drain(in0(t))
        compute(buf0, crows0, lrows0, ne0, no0, 0)
        issue(out0(t))

        @pl.when(t < ITEMS - 1)
        def _():
            issue(in0(t + 1))

        @pl.when(t > 0)
        def _():
            drain(out1(t - 1))
        drain(in1(t))
        compute(buf1, crows1, lrows1, ne1, no1, 1)
        issue(out1(t))

        @pl.when(t < ITEMS - 1)
        def _():
            issue(in1(t + 1))
        return 0
    lax.fori_loop(0, ITEMS, step, 0)

    drain(out0(ITEMS - 1))
    drain(out1(ITEMS - 1))


def kernel(example, label, mus_label, mus_class, noise_even, noise_odd):
    mesh = plsc.VectorSubcoreMesh(core_axis_name="c", subcore_axis_name="s",
                                  num_cores=NC, num_subcores=NS)
    call = pl.kernel(
        _sc_body, mesh=mesh,
        out_type=jax.ShapeDtypeStruct((S, SEQ_LEN, FEAT), jnp.float32),
        scratch_types=[
            pltpu.VMEM((ITEMS, N_EX), jnp.int32),
            pltpu.VMEM((ITEMS, N_PAIRS), jnp.int32),
            pltpu.VMEM((ITEMS, 2, 64), jnp.int32),
            pltpu.VMEM((ITEMS, 2, 32), jnp.int32),
            pltpu.VMEM((64, D), jnp.float32),
            pltpu.VMEM((32, D), jnp.float32),
            pltpu.VMEM((32, D), jnp.float32),
            pltpu.VMEM((32, D), jnp.float32),
            pltpu.VMEM((64, D), jnp.float32),
            pltpu.VMEM((32, D), jnp.float32),
            pltpu.VMEM((32, D), jnp.float32),
            pltpu.VMEM((32, D), jnp.float32),
            pltpu.VMEM((96, D), jnp.float32),
            pltpu.VMEM((96, D), jnp.float32),
            pltpu.VMEM((96, D), jnp.float32),
            pltpu.SemaphoreType.DMA,
            pltpu.SemaphoreType.DMA,
            pltpu.SemaphoreType.DMA,
            pltpu.SemaphoreType.DMA,
        ],
    )
    return call(example.astype(jnp.int32), label.astype(jnp.int32),
                mus_label, mus_class, noise_even, noise_odd)


# speedup vs baseline: 54.0074x; 3.1597x over previous
import jax
import jax.numpy as jnp
import numpy as np
from jax import lax
from jax.experimental import pallas as pl
from jax.experimental.pallas import tpu as pltpu
from jax.experimental.pallas import tpu_sc as plsc

S = 1024
NMAX = 64
D = 128
N_PAIRS = 63
N_EX = 2 * N_PAIRS
SEQ_LEN = 188
FEAT = 2 * NMAX + D
EPS = 0.1
A = float(1.0 / np.sqrt(1.0 + EPS * EPS))
B = float(A * EPS / np.sqrt(D))

NC, NS = 2, 16
NW = NC * NS
LANES = 16
NVH = D // LANES
SB = S // NW
NSETS = 3
NSTEP = N_PAIRS // NSETS


def _sc_body(example_h, label_h, mus_label_h, mus_class_h, ne_h, no_h,
             out_h, et, lt, zbuf, sets, isems, osems):
    wid = lax.axis_index("s") * NC + lax.axis_index("c")
    s0 = SB * wid

    def zrow(r, _):
        for v in range(NVH):
            zbuf[r, pl.ds(LANES * v, LANES)] = jnp.zeros((LANES,), jnp.float32)
        return 0
    lax.fori_loop(0, SB, zrow, 0)

    pltpu.sync_copy(example_h.at[wid], et)
    pltpu.sync_copy(label_h.at[wid], lt)

    def in_copies(p, k):
        ce, co, lr, ne, no = sets[k]
        sem = isems[k]
        return (
            pltpu.make_async_copy(mus_class_h.at[et.at[2 * p]], ce, sem),
            pltpu.make_async_copy(mus_class_h.at[et.at[2 * p + 1]], co, sem),
            pltpu.make_async_copy(mus_label_h.at[lt.at[p]], lr, sem),
            pltpu.make_async_copy(ne_h.at[p, pl.ds(s0, SB)], ne, sem),
            pltpu.make_async_copy(no_h.at[p, pl.ds(s0, SB)], no, sem),
        )

    def out_parts(p, k):
        ce, co, lr, _, _ = sets[k]
        sem = osems[k]
        parts = []
        for t, src in ((3 * p, ce), (3 * p + 1, co)):
            parts.append(pltpu.make_async_copy(
                zbuf, out_h.at[t, pl.ds(s0, SB), pl.ds(0, D)], sem))
            parts.append(pltpu.make_async_copy(
                src, out_h.at[t, pl.ds(s0, SB), pl.ds(D, D)], sem))
        return parts

    def lab_parts(p, k):
        _, _, lr, _, _ = sets[k]
        sem = osems[k]
        t = 3 * p + 2
        return (
            pltpu.make_async_copy(zbuf, out_h.at[t, pl.ds(s0, SB), pl.ds(0, D)], sem),
            pltpu.make_async_copy(lr, out_h.at[t, pl.ds(s0, SB), pl.ds(D, D)], sem),
        )

    def issue_out(p, k):
        for c in out_parts(p, k):
            c.start()

        @pl.when(p < N_PAIRS - 1)
        def _():
            for c in lab_parts(p, k):
                c.start()

    def wait_out(p, k):
        for c in out_parts(p, k):
            c.wait()

        @pl.when(p < N_PAIRS - 1)
        def _():
            for c in lab_parts(p, k):
                c.wait()

    def issue(copies):
        for c in copies:
            c.start()

    def drain(copies):
        for c in copies:
            c.wait()

    def compute(k):
        ce, co, _, ne, no = sets[k]

        def row(i, _):
            for v in range(NVH):
                sv = pl.ds(LANES * v, LANES)
                ce[i, sv] = A * ce[i, sv] + B * ne[i, sv]
                co[i, sv] = A * co[i, sv] + B * no[i, sv]
            return 0
        lax.fori_loop(0, SB, row, 0)

    for k in range(NSETS):
        issue(in_copies(k, k))

    def step(u, _):
        j = NSETS * u
        for k in range(NSETS):
            drain(in_copies(j + k, k))
            compute(k)
            issue_out(j + k, k)
        for k in range(NSETS):
            @pl.when(j + NSETS + k < N_PAIRS)
            def _():
                wait_out(j + k, k)
                issue(in_copies(j + NSETS + k, k))
        return 0
    lax.fori_loop(0, NSTEP, step, 0)

    for k in range(NSETS):
        wait_out(N_PAIRS - NSETS + k, k)


def kernel(example, label, mus_label, mus_class, noise_even, noise_odd):
    mesh = plsc.VectorSubcoreMesh(core_axis_name="c", subcore_axis_name="s",
                                  num_cores=NC, num_subcores=NS)
    buf = lambda: pltpu.VMEM((SB, D), jnp.float32)
    call = pl.kernel(
        _sc_body, mesh=mesh,
        out_type=jax.ShapeDtypeStruct((SEQ_LEN, S, FEAT), jnp.float32),
        scratch_types=[
            pltpu.VMEM((N_EX, SB), jnp.int32),
            pltpu.VMEM((N_PAIRS, SB), jnp.int32),
            buf(),
            [[buf() for _ in range(5)] for _ in range(NSETS)],
            [pltpu.SemaphoreType.DMA for _ in range(NSETS)],
            [pltpu.SemaphoreType.DMA for _ in range(NSETS)],
        ],
    )
    ne_t = jnp.transpose(noise_even, (1, 0, 2))
    no_t = jnp.transpose(noise_odd, (1, 0, 2))
    exw = jnp.transpose(example.astype(jnp.int32), (1, 0)) \
             .reshape(N_EX, NW, SB).transpose(1, 0, 2)
    lbw = jnp.transpose(label.astype(jnp.int32), (1, 0)) \
             .reshape(N_PAIRS, NW, SB).transpose(1, 0, 2)
    out_t = call(exw, lbw, mus_label, mus_class, ne_t, no_t)
    return jnp.transpose(out_t, (1, 0, 2))
